# baseline (speedup 1.0000x reference)
import functools

import jax
import jax.numpy as jnp
from jax import lax
from jax.experimental import pallas as pl
from jax.experimental.pallas import tpu as pltpu
from jax.experimental.pallas import tpu_sc as plsc

_B, _C, _N, _K = 8, 64, 16384, 64
_D_VAR, _D_DIST = 0.5, 1.5
_PAR_VAR, _PAR_DIST, _PAR_REG = 1.0, 1.0, 0.001

_W = 80
_BH = _B // 2
_PTS = _BH * _N
_NW = 32
_PPW = _PTS // _NW
_CH = 128
_NCH = _PPW // _CH
_T = 8192
_AR = _BH * _K
_NB = 3


def _prep_body(pred_ref, out_ref):
    x = pred_ref[0]
    m = jnp.sum(x, axis=0) * (1.0 / _C)
    d = x - m[None, :]
    nrm = jnp.sqrt(jnp.sum(d * d, axis=0))
    v = jnp.maximum(nrm - _D_VAR, 0.0)
    v = v * v
    h = _C // 2
    out_ref[0, :, 0:h] = x[0:h].T
    r = lax.broadcasted_iota(jnp.int32, (h, h), 0)
    c = lax.broadcasted_iota(jnp.int32, (h, h), 1)
    eye = (r == c).astype(jnp.float32)
    out_ref[0, :, h:_C] = lax.dot_general(
        x[h:_C], eye, (((0,), (0,)), ((), ())),
        preferred_element_type=jnp.float32,
        precision=lax.Precision.HIGHEST)
    col = lax.broadcasted_iota(jnp.int32, (_T, 16), 1)
    extra = jnp.where(col == 0, v[:, None],
                      jnp.where(col == 1, 1.0, 0.0))
    out_ref[0, :, _C:_W] = extra


def _prep_part(prediction, b0):
    return pl.pallas_call(
        _prep_body,
        grid=(_BH, _N // _T),
        in_specs=[pl.BlockSpec((1, _C, _T),
                               lambda b, i, b0=b0: (b + b0, 0, i))],
        out_specs=pl.BlockSpec((1, _T, _W), lambda b, i: (b, i, 0)),
        out_shape=jax.ShapeDtypeStruct((_BH, _N, _W), jnp.float32),
    )(prediction)


def _segsum_body(pred_ext, lab2, out, buf0, buf1, buf2, idx_all, acc,
                 gs0, gs1, gs2, as0, as1, as2):
    bufs, gsem, asem = [buf0, buf1, buf2], [gs0, gs1, gs2], [as0, as1, as2]
    cid = lax.axis_index("c")
    sid = lax.axis_index("s")
    wid = sid * 2 + cid

    zrows = _AR // 16
    zvec = jnp.zeros((16,), jnp.float32)

    def zero_row(r, carry):
        for j in range(_W // 16):
            buf0[r, pl.ds(j * 16, 16)] = zvec
        return carry

    lax.fori_loop(0, zrows, zero_row, 0)
    pltpu.sync_copy(buf0.at[pl.ds(0, zrows)], acc.at[pl.ds(sid * zrows, zrows)])
    plsc.subcore_barrier()

    pltpu.sync_copy(lab2.at[wid], idx_all)

    base = wid * _PPW

    def gdesc(i):
        return pltpu.make_async_copy(
            pred_ext.at[pl.ds(base + i * _CH, _CH)], bufs[i % _NB],
            gsem[i % _NB])

    def adesc(i):
        return pltpu.make_async_copy(
            bufs[i % _NB], acc.at[idx_all.at[i]], asem[i % _NB])

    for i in range(_NB - 1):
        gdesc(i).start()
    for i in range(_NCH):
        if i + _NB - 1 < _NCH:
            if i - 1 >= 0:
                adesc(i - 1).wait()
            gdesc(i + _NB - 1).start()
        gdesc(i).wait()
        adesc(i).start(add=True)
    for i in range(_NCH - _NB, _NCH):
        adesc(i).wait()
    plsc.subcore_barrier()

    pltpu.sync_copy(acc.at[pl.ds(sid * zrows, zrows)],
                    out.at[cid, pl.ds(sid * zrows, zrows)])


@functools.cache
def _segsum():
    return pl.kernel(
        _segsum_body,
        out_type=jax.ShapeDtypeStruct((2, _AR, _W), jnp.float32),
        mesh=plsc.VectorSubcoreMesh(core_axis_name="c", subcore_axis_name="s"),
        scratch_types=[
            pltpu.VMEM((_CH, _W), jnp.float32),
            pltpu.VMEM((_CH, _W), jnp.float32),
            pltpu.VMEM((_CH, _W), jnp.float32),
            pltpu.VMEM((_NCH, _CH), jnp.int32),
            pltpu.VMEM_SHARED((_AR, _W), jnp.float32),
            pltpu.SemaphoreType.DMA, pltpu.SemaphoreType.DMA,
            pltpu.SemaphoreType.DMA, pltpu.SemaphoreType.DMA,
            pltpu.SemaphoreType.DMA, pltpu.SemaphoreType.DMA,
        ],
    )


def _finish_body(p0_ref, p1_ref, out_ref):
    total = jnp.float32(0.0)
    for p_ref in (p0_ref, p1_ref):
        a = p_ref[0] + p_ref[1]
        for b in range(_BH):
            blk = a[b * _K:(b + 1) * _K, :]
            sums = blk[:, 0:_C]
            segv = blk[:, _C:_C + 1]
            counts = blk[:, _C + 1:_C + 2]
            center = sums / counts
            sq = jnp.sum(center * center, axis=0)
            l_reg = jnp.sum(jnp.sqrt(sq)) / _K
            var_b = jnp.sum(segv / counts) / _K
            g = lax.dot_general(center, center, (((0,), (0,)), ((), ())),
                                preferred_element_type=jnp.float32,
                                precision=lax.Precision.HIGHEST)
            dist = -2.0 * g + sq[:, None] + sq[None, :]
            dm = jnp.sqrt(jnp.maximum(dist, 0.0))
            hinge = jnp.maximum(2.0 * _D_DIST - dm, 0.0)
            dist_b = jnp.sum(hinge * hinge) / (2.0 * _K * (_K - 1.0 + 1e-16))
            total = total + (_PAR_VAR * var_b + _PAR_DIST * dist_b
                             + _PAR_REG * l_reg)
    out_ref[...] = jnp.reshape(total, (1, 1))


def _finish(p0, p1):
    return pl.pallas_call(
        _finish_body,
        out_shape=jax.ShapeDtypeStruct((1, 1), jnp.float32),
    )(p0, p1)


def kernel(prediction, label):
    lab = label.astype(jnp.int32)
    offs = (jnp.arange(_BH, dtype=jnp.int32) * _K)[:, None]
    lab0 = (lab[:_BH] + offs).reshape(_NW, _NCH, _CH)
    lab1 = (lab[_BH:] + offs).reshape(_NW, _NCH, _CH)
    pe0 = _prep_part(prediction, 0)
    p0 = _segsum()(pe0.reshape(_PTS, _W), lab0)
    pe1 = _prep_part(prediction, _BH)
    p1 = _segsum()(pe1.reshape(_PTS, _W), lab1)
    return _finish(p0, p1)[0, 0]


# speedup vs baseline: 1.4032x; 1.4032x over previous
import functools

import jax
import jax.numpy as jnp
from jax import lax
from jax.experimental import pallas as pl
from jax.experimental.pallas import tpu as pltpu
from jax.experimental.pallas import tpu_sc as plsc

_B, _C, _N, _K = 8, 64, 16384, 64
_D_VAR, _D_DIST = 0.5, 1.5
_PAR_VAR, _PAR_DIST, _PAR_REG = 1.0, 1.0, 0.001

_W = 80
_NS = 4
_BH = _B // _NS
_PTS = _BH * _N
_NW = 32
_PPW = _PTS // _NW
_CH = 128
_NCH = _PPW // _CH
_T = 16384
_AR = _BH * _K
_NB = 3


def _prep_body(pred_ref, out_ref):
    x = pred_ref[0]
    m = jnp.sum(x, axis=0) * (1.0 / _C)
    d = x - m[None, :]
    nrm = jnp.sqrt(jnp.sum(d * d, axis=0))
    v = jnp.maximum(nrm - _D_VAR, 0.0)
    v = v * v
    out_ref[0, :, 0:_C] = x.T
    col = lax.broadcasted_iota(jnp.int32, (_T, 16), 1)
    extra = jnp.where(col == 0, v[:, None],
                      jnp.where(col == 1, 1.0, 0.0))
    out_ref[0, :, _C:_W] = extra


def _prep_part(prediction, b0):
    return pl.pallas_call(
        _prep_body,
        grid=(_BH, _N // _T),
        in_specs=[pl.BlockSpec((1, _C, _T),
                               lambda b, i, b0=b0: (b + b0, 0, i))],
        out_specs=pl.BlockSpec((1, _T, _W), lambda b, i: (b, i, 0)),
        out_shape=jax.ShapeDtypeStruct((_BH, _N, _W), jnp.float32),
    )(prediction)


def _segsum_body(pred_ext, lab2, out, buf0, buf1, buf2, idx_all, acc,
                 gs0, gs1, gs2, as0, as1, as2):
    bufs, gsem, asem = [buf0, buf1, buf2], [gs0, gs1, gs2], [as0, as1, as2]
    cid = lax.axis_index("c")
    sid = lax.axis_index("s")
    wid = sid * 2 + cid

    zrows = _AR // 16
    zvec = jnp.zeros((16,), jnp.float32)

    def zero_row(r, carry):
        for j in range(_W // 16):
            buf0[r, pl.ds(j * 16, 16)] = zvec
        return carry

    lax.fori_loop(0, zrows, zero_row, 0)
    pltpu.sync_copy(buf0.at[pl.ds(0, zrows)], acc.at[pl.ds(sid * zrows, zrows)])
    plsc.subcore_barrier()

    pltpu.sync_copy(lab2.at[wid], idx_all)

    base = wid * _PPW

    def gdesc(i):
        return pltpu.make_async_copy(
            pred_ext.at[pl.ds(base + i * _CH, _CH)], bufs[i % _NB],
            gsem[i % _NB])

    def adesc(i):
        return pltpu.make_async_copy(
            bufs[i % _NB], acc.at[idx_all.at[i]], asem[i % _NB])

    for i in range(_NB - 1):
        gdesc(i).start()
    for i in range(_NCH):
        if i + _NB - 1 < _NCH:
            if i - 1 >= 0:
                adesc(i - 1).wait()
            gdesc(i + _NB - 1).start()
        gdesc(i).wait()
        adesc(i).start(add=True)
    for i in range(_NCH - _NB, _NCH):
        adesc(i).wait()
    plsc.subcore_barrier()

    pltpu.sync_copy(acc.at[pl.ds(sid * zrows, zrows)],
                    out.at[cid, pl.ds(sid * zrows, zrows)])


@functools.cache
def _segsum():
    return pl.kernel(
        _segsum_body,
        out_type=jax.ShapeDtypeStruct((2, _AR, _W), jnp.float32),
        mesh=plsc.VectorSubcoreMesh(core_axis_name="c", subcore_axis_name="s"),
        scratch_types=[
            pltpu.VMEM((_CH, _W), jnp.float32),
            pltpu.VMEM((_CH, _W), jnp.float32),
            pltpu.VMEM((_CH, _W), jnp.float32),
            pltpu.VMEM((_NCH, _CH), jnp.int32),
            pltpu.VMEM_SHARED((_AR, _W), jnp.float32),
            pltpu.SemaphoreType.DMA, pltpu.SemaphoreType.DMA,
            pltpu.SemaphoreType.DMA, pltpu.SemaphoreType.DMA,
            pltpu.SemaphoreType.DMA, pltpu.SemaphoreType.DMA,
        ],
    )


def _finish_body(*refs):
    p_refs, out_ref = refs[:-1], refs[-1]
    total = jnp.float32(0.0)
    for p_ref in p_refs:
        a = p_ref[0] + p_ref[1]
        for b in range(_BH):
            blk = a[b * _K:(b + 1) * _K, :]
            sums = blk[:, 0:_C]
            segv = blk[:, _C:_C + 1]
            counts = blk[:, _C + 1:_C + 2]
            center = sums / counts
            sq = jnp.sum(center * center, axis=0)
            l_reg = jnp.sum(jnp.sqrt(sq)) / _K
            var_b = jnp.sum(segv / counts) / _K
            g = lax.dot_general(center, center, (((0,), (0,)), ((), ())),
                                preferred_element_type=jnp.float32,
                                precision=lax.Precision.HIGHEST)
            dist = -2.0 * g + sq[:, None] + sq[None, :]
            dm = jnp.sqrt(jnp.maximum(dist, 0.0))
            hinge = jnp.maximum(2.0 * _D_DIST - dm, 0.0)
            dist_b = jnp.sum(hinge * hinge) / (2.0 * _K * (_K - 1.0 + 1e-16))
            total = total + (_PAR_VAR * var_b + _PAR_DIST * dist_b
                             + _PAR_REG * l_reg)
    out_ref[...] = jnp.reshape(total, (1, 1))


def _finish(parts):
    return pl.pallas_call(
        _finish_body,
        out_shape=jax.ShapeDtypeStruct((1, 1), jnp.float32),
    )(*parts)


def kernel(prediction, label):
    lab = label.astype(jnp.int32)
    offs = (jnp.arange(_BH, dtype=jnp.int32) * _K)[:, None]
    parts = []
    for s in range(_NS):
        labs = (lab[s * _BH:(s + 1) * _BH] + offs).reshape(_NW, _NCH, _CH)
        pes = _prep_part(prediction, s * _BH)
        parts.append(_segsum()(pes.reshape(_PTS, _W), labs))
    return _finish(parts)[0, 0]


# speedup vs baseline: 1.4864x; 1.0593x over previous
import functools

import jax
import jax.numpy as jnp
from jax import lax
from jax.experimental import pallas as pl
from jax.experimental.pallas import tpu as pltpu
from jax.experimental.pallas import tpu_sc as plsc

_B, _C, _N, _K = 8, 64, 16384, 64
_D_VAR, _D_DIST = 0.5, 1.5
_PAR_VAR, _PAR_DIST, _PAR_REG = 1.0, 1.0, 0.001

_W = 80
_SPLITS = (5, 3)
_NW = 32
_CH = 128
_NB = 3


def _prep_body(pred_ref, out_ref):
    x = pred_ref[0]
    m = jnp.sum(x, axis=0) * (1.0 / _C)
    d = x - m[None, :]
    nrm = jnp.sqrt(jnp.sum(d * d, axis=0))
    v = jnp.maximum(nrm - _D_VAR, 0.0)
    v = v * v
    out_ref[0, :, 0:_C] = x.T
    col = lax.broadcasted_iota(jnp.int32, (_N, 16), 1)
    extra = jnp.where(col == 0, v[:, None],
                      jnp.where(col == 1, 1.0, 0.0))
    out_ref[0, :, _C:_W] = extra


def _prep_part(prediction, b0, nb):
    return pl.pallas_call(
        _prep_body,
        grid=(nb, 1),
        in_specs=[pl.BlockSpec((1, _C, _N),
                               lambda b, i, b0=b0: (b + b0, 0, i))],
        out_specs=pl.BlockSpec((1, _N, _W), lambda b, i: (b, i, 0)),
        out_shape=jax.ShapeDtypeStruct((nb, _N, _W), jnp.float32),
    )(prediction)


def _make_segsum_body(nb):
    ppw = nb * _N // _NW
    nch = ppw // _CH
    ar = nb * _K
    nz = ar // 32

    def _segsum_body(pred_ext, lab2, out, buf0, buf1, buf2, idx_all, acc,
                     gs0, gs1, gs2, as0, as1, as2):
        bufs = [buf0, buf1, buf2]
        gsem, asem = [gs0, gs1, gs2], [as0, as1, as2]
        cid = lax.axis_index("c")
        sid = lax.axis_index("s")
        wid = sid * 2 + cid

        zvec = jnp.zeros((16,), jnp.float32)

        def zero_row(r, carry):
            for j in range(_W // 16):
                buf0[r, pl.ds(j * 16, 16)] = zvec
            return carry

        lax.fori_loop(0, 32, zero_row, 0)

        @pl.when(sid < nz)
        def _zero_acc():
            pltpu.sync_copy(buf0.at[pl.ds(0, 32)],
                            acc.at[pl.ds(sid * 32, 32)])

        plsc.subcore_barrier()

        pltpu.sync_copy(lab2.at[wid], idx_all)

        base = wid * ppw

        def gdesc(i):
            return pltpu.make_async_copy(
                pred_ext.at[pl.ds(base + i * _CH, _CH)], bufs[i % _NB],
                gsem[i % _NB])

        def adesc(i):
            return pltpu.make_async_copy(
                bufs[i % _NB], acc.at[idx_all.at[i]], asem[i % _NB])

        for i in range(_NB - 1):
            gdesc(i).start()
        for i in range(nch):
            if i + _NB - 1 < nch:
                if i - 1 >= 0:
                    adesc(i - 1).wait()
                gdesc(i + _NB - 1).start()
            gdesc(i).wait()
            adesc(i).start(add=True)
        for i in range(nch - _NB, nch):
            adesc(i).wait()
        plsc.subcore_barrier()

        @pl.when(sid < nz)
        def _write_out():
            pltpu.sync_copy(acc.at[pl.ds(sid * 32, 32)],
                            out.at[cid, pl.ds(sid * 32, 32)])

    return _segsum_body


@functools.cache
def _segsum(nb):
    ppw = nb * _N // _NW
    nch = ppw // _CH
    ar = nb * _K
    return pl.kernel(
        _make_segsum_body(nb),
        out_type=jax.ShapeDtypeStruct((2, ar, _W), jnp.float32),
        mesh=plsc.VectorSubcoreMesh(core_axis_name="c", subcore_axis_name="s"),
        scratch_types=[
            pltpu.VMEM((_CH, _W), jnp.float32),
            pltpu.VMEM((_CH, _W), jnp.float32),
            pltpu.VMEM((_CH, _W), jnp.float32),
            pltpu.VMEM((nch, _CH), jnp.int32),
            pltpu.VMEM_SHARED((ar, _W), jnp.float32),
            pltpu.SemaphoreType.DMA, pltpu.SemaphoreType.DMA,
            pltpu.SemaphoreType.DMA, pltpu.SemaphoreType.DMA,
            pltpu.SemaphoreType.DMA, pltpu.SemaphoreType.DMA,
        ],
    )


def _finish_body(*refs):
    p_refs, out_ref = refs[:-1], refs[-1]
    total = jnp.float32(0.0)
    for p_ref in p_refs:
        a = p_ref[0] + p_ref[1]
        nb = a.shape[0] // _K
        for b in range(nb):
            blk = a[b * _K:(b + 1) * _K, :]
            sums = blk[:, 0:_C]
            segv = blk[:, _C:_C + 1]
            counts = blk[:, _C + 1:_C + 2]
            center = sums / counts
            sq = jnp.sum(center * center, axis=0)
            l_reg = jnp.sum(jnp.sqrt(sq)) / _K
            var_b = jnp.sum(segv / counts) / _K
            g = lax.dot_general(center, center, (((0,), (0,)), ((), ())),
                                preferred_element_type=jnp.float32,
                                precision=lax.Precision.HIGHEST)
            dist = -2.0 * g + sq[:, None] + sq[None, :]
            dm = jnp.sqrt(jnp.maximum(dist, 0.0))
            hinge = jnp.maximum(2.0 * _D_DIST - dm, 0.0)
            dist_b = jnp.sum(hinge * hinge) / (2.0 * _K * (_K - 1.0 + 1e-16))
            total = total + (_PAR_VAR * var_b + _PAR_DIST * dist_b
                             + _PAR_REG * l_reg)
    out_ref[...] = jnp.reshape(total, (1, 1))


def _finish(parts):
    return pl.pallas_call(
        _finish_body,
        out_shape=jax.ShapeDtypeStruct((1, 1), jnp.float32),
    )(*parts)


def kernel(prediction, label):
    lab = label.astype(jnp.int32)
    parts = []
    b0 = 0
    for nb in _SPLITS:
        offs = (jnp.arange(nb, dtype=jnp.int32) * _K)[:, None]
        nch = nb * _N // _NW // _CH
        labs = (lab[b0:b0 + nb] + offs).reshape(_NW, nch, _CH)
        pes = _prep_part(prediction, b0, nb)
        parts.append(_segsum(nb)(pes.reshape(nb * _N, _W), labs))
        b0 += nb
    return _finish(parts)[0, 0]
